# baseline (speedup 1.0000x reference)
import functools

import jax
import jax.numpy as jnp
from jax import lax
from jax.experimental import pallas as pl
from jax.experimental.pallas import tpu as pltpu
from jax.experimental.pallas import tpu_sc as plsc

N_VIEWS = 200
N_POINTS = 10000
NNZ = 320000
D = 128

NC = 2
NS = 16
NW = NC * NS
PER_W = NNZ // NW
K = 80
NCH = PER_W // K

CW = 16

NPPAD = 10240
RPT = NPPAD // NS
NVPAD = 256
RVT = 32

_mesh = plsc.VectorSubcoreMesh(
    core_axis_name="c", subcore_axis_name="s", num_cores=NC, num_subcores=NS)


def _zero_rows(zbuf, nrows, ncols16):
    z = jnp.zeros((16,), jnp.float32)

    def body(i, _):
        for c in range(ncols16):
            zbuf[i, pl.ds(16 * c, 16)] = z
        return 0

    lax.fori_loop(0, nrows, body, 0)


VR = 3
IDXR = 6


def _sc_segsum_body(values_hbm, colidx_hbm, rowidx_hbm,
                    sumcol_hbm, cntcol_hbm, sumrow_hbm, cntrow_hbm,
                    shared_col, shared_cntc, shared_row, shared_cntr,
                    idxc_v, idxr_v, vals_v, ones_v,
                    semic, semir, seml, sems):
    cid = lax.axis_index("c")
    sid = lax.axis_index("s")
    wid = sid * NC + cid
    base = wid * PER_W

    zsrc = vals_v.at[0]
    _zero_rows(zsrc, K, D // 16)
    _zero_rows(ones_v, K, 1)
    for t in range(RPT // K):
        off = sid * RPT + t * K
        pltpu.sync_copy(zsrc, shared_col.at[pl.ds(off, K)])
        pltpu.sync_copy(ones_v, shared_cntc.at[pl.ds(off, K)])

    @pl.when(sid < 8)
    def _():
        off = sid * RVT
        pltpu.sync_copy(zsrc.at[pl.ds(0, RVT)], shared_row.at[pl.ds(off, RVT)])
        pltpu.sync_copy(ones_v.at[pl.ds(0, RVT)], shared_cntr.at[pl.ds(off, RVT)])

    plsc.subcore_barrier()

    one = jnp.full((16,), 1.0, jnp.float32)

    def fill_ones(i, _):
        ones_v[i, :] = one
        return 0

    lax.fori_loop(0, K, fill_ones, 0)

    for p in range(2):
        pltpu.async_copy(colidx_hbm.at[wid, p], idxc_v.at[p], semic.at[p])
        pltpu.async_copy(rowidx_hbm.at[wid, p], idxr_v.at[p], semir.at[p])

    def step(j, _):
        bl = j % VR
        bi = j % IDXR

        @pl.when(j >= VR)
        def _():
            bio = (j - VR) % IDXR
            pltpu.make_async_copy(vals_v.at[bl],
                                  shared_col.at[idxc_v.at[bio]],
                                  sems.at[bl]).wait()
            pltpu.make_async_copy(vals_v.at[bl],
                                  shared_row.at[idxr_v.at[bio]],
                                  sems.at[bl]).wait()
            pltpu.make_async_copy(ones_v, shared_cntc.at[idxc_v.at[bio]],
                                  sems.at[bl]).wait()
            pltpu.make_async_copy(ones_v, shared_cntr.at[idxr_v.at[bio]],
                                  sems.at[bl]).wait()

        pltpu.async_copy(values_hbm.at[pl.ds(base + j * K, K)],
                         vals_v.at[bl], seml.at[bl])

        @pl.when(j + 2 < NCH)
        def _():
            b2 = (j + 2) % IDXR
            pltpu.async_copy(colidx_hbm.at[wid, j + 2], idxc_v.at[b2],
                             semic.at[b2])
            pltpu.async_copy(rowidx_hbm.at[wid, j + 2], idxr_v.at[b2],
                             semir.at[b2])

        pltpu.make_async_copy(colidx_hbm.at[wid, j], idxc_v.at[bi],
                              semic.at[bi]).wait()
        pltpu.make_async_copy(rowidx_hbm.at[wid, j], idxr_v.at[bi],
                              semir.at[bi]).wait()
        pltpu.make_async_copy(values_hbm.at[pl.ds(base + j * K, K)],
                              vals_v.at[bl], seml.at[bl]).wait()

        pltpu.async_copy(vals_v.at[bl], shared_col.at[idxc_v.at[bi]],
                         sems.at[bl], add=True)
        pltpu.async_copy(vals_v.at[bl], shared_row.at[idxr_v.at[bi]],
                         sems.at[bl], add=True)
        pltpu.async_copy(ones_v, shared_cntc.at[idxc_v.at[bi]],
                         sems.at[bl], add=True)
        pltpu.async_copy(ones_v, shared_cntr.at[idxr_v.at[bi]],
                         sems.at[bl], add=True)
        return 0

    lax.fori_loop(0, NCH, step, 0)

    for d in range(VR):
        j = NCH - VR + d
        bl = j % VR
        pltpu.make_async_copy(vals_v.at[bl], shared_col.at[idxc_v.at[0]],
                              sems.at[bl]).wait()
        pltpu.make_async_copy(vals_v.at[bl], shared_row.at[idxr_v.at[0]],
                              sems.at[bl]).wait()
        pltpu.make_async_copy(ones_v, shared_cntc.at[idxc_v.at[0]],
                              sems.at[bl]).wait()
        pltpu.make_async_copy(ones_v, shared_cntr.at[idxr_v.at[0]],
                              sems.at[bl]).wait()

    plsc.subcore_barrier()

    for t in range(RPT // 128):
        off = sid * RPT + t * 128
        pltpu.sync_copy(shared_col.at[pl.ds(off, 128)],
                        sumcol_hbm.at[cid, pl.ds(off, 128)])
    for t in range(RPT // 128):
        off = sid * RPT + t * 128
        pltpu.sync_copy(shared_cntc.at[pl.ds(off, 128)],
                        cntcol_hbm.at[cid, pl.ds(off, 128)])

    @pl.when(sid < 8)
    def _():
        off = sid * RVT
        pltpu.sync_copy(shared_row.at[pl.ds(off, RVT)],
                        sumrow_hbm.at[cid, pl.ds(off, RVT)])
        pltpu.sync_copy(shared_cntr.at[pl.ds(off, RVT)],
                        cntrow_hbm.at[cid, pl.ds(off, RVT)])


_sc_segsum = pl.kernel(
    _sc_segsum_body,
    out_type=[
        jax.ShapeDtypeStruct((NC, NPPAD, D), jnp.float32),
        jax.ShapeDtypeStruct((NC, NPPAD, CW), jnp.float32),
        jax.ShapeDtypeStruct((NC, NVPAD, D), jnp.float32),
        jax.ShapeDtypeStruct((NC, NVPAD, CW), jnp.float32),
    ],
    mesh=_mesh,
    scratch_types=[
        pltpu.VMEM_SHARED((NPPAD, D), jnp.float32),
        pltpu.VMEM_SHARED((NPPAD, CW), jnp.float32),
        pltpu.VMEM_SHARED((NVPAD, D), jnp.float32),
        pltpu.VMEM_SHARED((NVPAD, CW), jnp.float32),
        pltpu.VMEM((IDXR, K), jnp.int32),
        pltpu.VMEM((IDXR, K), jnp.int32),
        pltpu.VMEM((VR, K, D), jnp.float32),
        pltpu.VMEM((K, CW), jnp.float32),
        pltpu.SemaphoreType.DMA((IDXR,)),
        pltpu.SemaphoreType.DMA((IDXR,)),
        pltpu.SemaphoreType.DMA((VR,)),
        pltpu.SemaphoreType.DMA((VR,)),
    ],
    compiler_params=pltpu.CompilerParams(use_tc_tiling_on_sc=False),
)


S = 5
SL_ROWS = NNZ // S
PER_WS = SL_ROWS // NW
NCH_S = PER_WS // K

NBUF = 5


def _sc_gather_body(tcol_hbm, colidx_hbm, out_hbm, idx_v, gbuf, semg, semw):
    cid = lax.axis_index("c")
    sid = lax.axis_index("s")
    wid = sid * NC + cid
    base = wid * PER_WS

    pltpu.sync_copy(colidx_hbm.at[wid], idx_v)

    def wave(w, _):
        j0 = w * NBUF
        @pl.when(w > 0)
        def _():
            for b in range(NBUF):
                pltpu.make_async_copy(
                    gbuf.at[b],
                    out_hbm.at[pl.ds(base + (j0 - NBUF + b) * K, K)],
                    semw.at[b]).wait()
        for b in range(NBUF):
            pltpu.async_copy(tcol_hbm.at[idx_v.at[j0 + b]], gbuf.at[b],
                             semg.at[b])
        for b in range(NBUF):
            pltpu.make_async_copy(tcol_hbm.at[idx_v.at[j0 + b]], gbuf.at[b],
                                  semg.at[b]).wait()
            pltpu.async_copy(gbuf.at[b],
                             out_hbm.at[pl.ds(base + (j0 + b) * K, K)],
                             semw.at[b])
        return 0

    lax.fori_loop(0, NCH_S // NBUF, wave, 0)

    for b in range(NBUF):
        pltpu.make_async_copy(
            gbuf.at[b],
            out_hbm.at[pl.ds(base + (NCH_S - NBUF + b) * K, K)],
            semw.at[b]).wait()


_sc_gather = pl.kernel(
    _sc_gather_body,
    out_type=jax.ShapeDtypeStruct((SL_ROWS, D), jnp.float32),
    mesh=_mesh,
    scratch_types=[
        pltpu.VMEM((NCH_S, K), jnp.int32),
        pltpu.VMEM((NBUF, K, D), jnp.float32),
        pltpu.SemaphoreType.DMA((NBUF,)),
        pltpu.SemaphoreType.DMA((NBUF,)),
    ],
)


BLK_P = 512
BLK_O = 12800


def _tc_tcol_body(sums_ref, cnts_ref, w_ref, b_ref, out_ref):
    s = sums_ref[0] + sums_ref[1]
    c = cnts_ref[0, :, 0:1] + cnts_ref[1, :, 0:1]
    mean = s / jnp.clip(c, 1.0)
    out_ref[...] = (jnp.dot(mean, w_ref[...] * 0.25,
                            preferred_element_type=jnp.float32)
                    + b_ref[...] * 0.25)


def _tc_tcol(sum_col, cnt_col, w_sp, b_sp):
    return pl.pallas_call(
        _tc_tcol_body,
        grid=(NPPAD // BLK_P,),
        in_specs=[
            pl.BlockSpec((NC, BLK_P, D), lambda i: (0, i, 0)),
            pl.BlockSpec((NC, BLK_P, CW), lambda i: (0, i, 0)),
            pl.BlockSpec((D, D), lambda i: (0, 0)),
            pl.BlockSpec((1, D), lambda i: (0, 0)),
        ],
        out_specs=pl.BlockSpec((BLK_P, D), lambda i: (i, 0)),
        out_shape=jax.ShapeDtypeStruct((NPPAD, D), jnp.float32),
    )(sum_col, cnt_col, w_sp, b_sp)


def _tc_trow_body(sums_ref, cnts_ref, wv_ref, bv_ref, wg_ref, bg_ref,
                  bp_ref, out_ref):
    rs = sums_ref[0] + sums_ref[1]
    rc = cnts_ref[0, :, 0:1] + cnts_ref[1, :, 0:1]
    view = (jnp.dot(rs / jnp.clip(rc, 1.0), wv_ref[...],
                    preferred_element_type=jnp.float32) + bv_ref[...])
    total = jnp.sum(rs, axis=0, keepdims=True)
    g = (jnp.dot(total * (1.0 / NNZ), wg_ref[...],
                 preferred_element_type=jnp.float32) + bg_ref[...])
    out_ref[...] = (view + g + bp_ref[...]) * 0.25


def _tc_trow(sum_row, cnt_row, w_view, b_view, w_glob, b_glob, b_proj):
    return pl.pallas_call(
        _tc_trow_body,
        out_shape=jax.ShapeDtypeStruct((NVPAD, D), jnp.float32),
    )(sum_row, cnt_row, w_view, b_view, w_glob, b_glob, b_proj)


def _tc_combine_body(vals_ref, g_ref, row_ref, trow_ref, w_ref, out_ref):
    r = row_ref[0, 0, :]
    iota = lax.broadcasted_iota(jnp.int32, (BLK_O, NVPAD), 1)
    oh = (iota == r[:, None]).astype(jnp.bfloat16)
    acc = jnp.dot(vals_ref[...].astype(jnp.bfloat16),
                  (w_ref[...] * 0.25).astype(jnp.bfloat16),
                  preferred_element_type=jnp.float32)
    acc += jnp.dot(oh, trow_ref[...].astype(jnp.bfloat16),
                   preferred_element_type=jnp.float32)
    out_ref[...] = acc + g_ref[...]


def _tc_combine_slice(s, values, g, row3, t_row, w_proj, prev_out):
    nblk = SL_ROWS // BLK_O
    off = s * nblk
    in_specs = [
        pl.BlockSpec((BLK_O, D), lambda i: (off + i, 0)),
        pl.BlockSpec((BLK_O, D), lambda i: (i, 0)),
        pl.BlockSpec((1, 1, BLK_O), lambda i: (off + i, 0, 0)),
        pl.BlockSpec((NVPAD, D), lambda i: (0, 0)),
        pl.BlockSpec((D, D), lambda i: (0, 0)),
    ]
    args = [values, g, row3, t_row, w_proj]
    kwargs = {}
    body = _tc_combine_body
    if prev_out is not None:
        in_specs.append(pl.BlockSpec(memory_space=pl.ANY))
        args.append(prev_out)
        kwargs["input_output_aliases"] = {5: 0}

        def body(v, gr, rr, tr, wr, _prev, out_ref):
            _tc_combine_body(v, gr, rr, tr, wr, out_ref)

    return pl.pallas_call(
        body,
        grid=(nblk,),
        in_specs=in_specs,
        out_specs=pl.BlockSpec((BLK_O, D), lambda i: (off + i, 0)),
        out_shape=jax.ShapeDtypeStruct((NNZ, D), jnp.float32),
        **kwargs,
    )(*args)


def kernel(values, row_idx, col_idx, W_sp, b_sp, W_view, b_view,
           W_glob, b_glob, W_proj, b_proj):
    col3 = col_idx.reshape(NW, NCH, K)
    row3 = row_idx.reshape(NW, NCH, K)

    sum_col, cnt_col, sum_row, cnt_row = _sc_segsum(values, col3, row3)
    t_col = _tc_tcol(sum_col, cnt_col, W_sp, b_sp.reshape(1, D))
    t_row = _tc_trow(sum_row, cnt_row, W_view, b_view.reshape(1, D),
                     W_glob, b_glob.reshape(1, D), b_proj.reshape(1, D))
    row3b = row_idx.reshape(NNZ // BLK_O, 1, BLK_O)
    cols = col_idx.reshape(S, NW, NCH_S, K)
    gs = [_sc_gather(t_col, cols[s]) for s in range(S)]
    out = None
    for s in range(S):
        out = _tc_combine_slice(s, values, gs[s], row3b, t_row, W_proj, out)
    return out


# speedup vs baseline: 1.0754x; 1.0754x over previous
import functools

import jax
import jax.numpy as jnp
from jax import lax
from jax.experimental import pallas as pl
from jax.experimental.pallas import tpu as pltpu
from jax.experimental.pallas import tpu_sc as plsc

N_VIEWS = 200
N_POINTS = 10000
NNZ = 320000
D = 128

NC = 2
NS = 16
NW = NC * NS
PER_W = NNZ // NW
K = 80
NCH = PER_W // K

CW = 16

NPPAD = 10240
RPT = NPPAD // NS
NVPAD = 256
RVT = 32

_mesh = plsc.VectorSubcoreMesh(
    core_axis_name="c", subcore_axis_name="s", num_cores=NC, num_subcores=NS)


def _zero_rows(zbuf, nrows, ncols16):
    z = jnp.zeros((16,), jnp.float32)

    def body(i, _):
        for c in range(ncols16):
            zbuf[i, pl.ds(16 * c, 16)] = z
        return 0

    lax.fori_loop(0, nrows, body, 0)


VR = 3
IDXR = 6


def _sc_segsum_body(values_hbm, colidx_hbm, rowidx_hbm,
                    sumcol_hbm, cntcol_hbm, sumrow_hbm, cntrow_hbm,
                    shared_col, shared_cntc, shared_row, shared_cntr,
                    idxc_v, idxr_v, vals_v, ones_v,
                    semic, semir, seml, sems):
    cid = lax.axis_index("c")
    sid = lax.axis_index("s")
    wid = sid * NC + cid
    base = wid * PER_W

    zsrc = vals_v.at[0]
    _zero_rows(zsrc, K, D // 16)
    _zero_rows(ones_v, K, 1)
    for t in range(RPT // K):
        off = sid * RPT + t * K
        pltpu.sync_copy(zsrc, shared_col.at[pl.ds(off, K)])
        pltpu.sync_copy(ones_v, shared_cntc.at[pl.ds(off, K)])

    @pl.when(sid < 8)
    def _():
        off = sid * RVT
        pltpu.sync_copy(zsrc.at[pl.ds(0, RVT)], shared_row.at[pl.ds(off, RVT)])
        pltpu.sync_copy(ones_v.at[pl.ds(0, RVT)], shared_cntr.at[pl.ds(off, RVT)])

    plsc.subcore_barrier()

    one = jnp.full((16,), 1.0, jnp.float32)

    def fill_ones(i, _):
        ones_v[i, :] = one
        return 0

    lax.fori_loop(0, K, fill_ones, 0)

    for p in range(2):
        pltpu.async_copy(colidx_hbm.at[wid, p], idxc_v.at[p], semic.at[p])
        pltpu.async_copy(rowidx_hbm.at[wid, p], idxr_v.at[p], semir.at[p])

    def step(j, _):
        bl = j % VR
        bi = j % IDXR

        @pl.when(j >= VR)
        def _():
            bio = (j - VR) % IDXR
            pltpu.make_async_copy(vals_v.at[bl],
                                  shared_col.at[idxc_v.at[bio]],
                                  sems.at[bl]).wait()
            pltpu.make_async_copy(vals_v.at[bl],
                                  shared_row.at[idxr_v.at[bio]],
                                  sems.at[bl]).wait()
            pltpu.make_async_copy(ones_v, shared_cntc.at[idxc_v.at[bio]],
                                  sems.at[bl]).wait()
            pltpu.make_async_copy(ones_v, shared_cntr.at[idxr_v.at[bio]],
                                  sems.at[bl]).wait()

        pltpu.async_copy(values_hbm.at[pl.ds(base + j * K, K)],
                         vals_v.at[bl], seml.at[bl])

        @pl.when(j + 2 < NCH)
        def _():
            b2 = (j + 2) % IDXR
            pltpu.async_copy(colidx_hbm.at[wid, j + 2], idxc_v.at[b2],
                             semic.at[b2])
            pltpu.async_copy(rowidx_hbm.at[wid, j + 2], idxr_v.at[b2],
                             semir.at[b2])

        pltpu.make_async_copy(colidx_hbm.at[wid, j], idxc_v.at[bi],
                              semic.at[bi]).wait()
        pltpu.make_async_copy(rowidx_hbm.at[wid, j], idxr_v.at[bi],
                              semir.at[bi]).wait()
        pltpu.make_async_copy(values_hbm.at[pl.ds(base + j * K, K)],
                              vals_v.at[bl], seml.at[bl]).wait()

        pltpu.async_copy(vals_v.at[bl], shared_col.at[idxc_v.at[bi]],
                         sems.at[bl], add=True)
        pltpu.async_copy(vals_v.at[bl], shared_row.at[idxr_v.at[bi]],
                         sems.at[bl], add=True)
        pltpu.async_copy(ones_v, shared_cntc.at[idxc_v.at[bi]],
                         sems.at[bl], add=True)
        pltpu.async_copy(ones_v, shared_cntr.at[idxr_v.at[bi]],
                         sems.at[bl], add=True)
        return 0

    lax.fori_loop(0, NCH, step, 0)

    for d in range(VR):
        j = NCH - VR + d
        bl = j % VR
        pltpu.make_async_copy(vals_v.at[bl], shared_col.at[idxc_v.at[0]],
                              sems.at[bl]).wait()
        pltpu.make_async_copy(vals_v.at[bl], shared_row.at[idxr_v.at[0]],
                              sems.at[bl]).wait()
        pltpu.make_async_copy(ones_v, shared_cntc.at[idxc_v.at[0]],
                              sems.at[bl]).wait()
        pltpu.make_async_copy(ones_v, shared_cntr.at[idxr_v.at[0]],
                              sems.at[bl]).wait()

    plsc.subcore_barrier()

    for t in range(RPT // 128):
        off = sid * RPT + t * 128
        pltpu.sync_copy(shared_col.at[pl.ds(off, 128)],
                        sumcol_hbm.at[cid, pl.ds(off, 128)])
    for t in range(RPT // 128):
        off = sid * RPT + t * 128
        pltpu.sync_copy(shared_cntc.at[pl.ds(off, 128)],
                        cntcol_hbm.at[cid, pl.ds(off, 128)])

    @pl.when(sid < 8)
    def _():
        off = sid * RVT
        pltpu.sync_copy(shared_row.at[pl.ds(off, RVT)],
                        sumrow_hbm.at[cid, pl.ds(off, RVT)])
        pltpu.sync_copy(shared_cntr.at[pl.ds(off, RVT)],
                        cntrow_hbm.at[cid, pl.ds(off, RVT)])


_sc_segsum = pl.kernel(
    _sc_segsum_body,
    out_type=[
        jax.ShapeDtypeStruct((NC, NPPAD, D), jnp.float32),
        jax.ShapeDtypeStruct((NC, NPPAD, CW), jnp.float32),
        jax.ShapeDtypeStruct((NC, NVPAD, D), jnp.float32),
        jax.ShapeDtypeStruct((NC, NVPAD, CW), jnp.float32),
    ],
    mesh=_mesh,
    scratch_types=[
        pltpu.VMEM_SHARED((NPPAD, D), jnp.float32),
        pltpu.VMEM_SHARED((NPPAD, CW), jnp.float32),
        pltpu.VMEM_SHARED((NVPAD, D), jnp.float32),
        pltpu.VMEM_SHARED((NVPAD, CW), jnp.float32),
        pltpu.VMEM((IDXR, K), jnp.int32),
        pltpu.VMEM((IDXR, K), jnp.int32),
        pltpu.VMEM((VR, K, D), jnp.float32),
        pltpu.VMEM((K, CW), jnp.float32),
        pltpu.SemaphoreType.DMA((IDXR,)),
        pltpu.SemaphoreType.DMA((IDXR,)),
        pltpu.SemaphoreType.DMA((VR,)),
        pltpu.SemaphoreType.DMA((VR,)),
    ],
    compiler_params=pltpu.CompilerParams(use_tc_tiling_on_sc=False),
)


S = 5
SL_ROWS = NNZ // S
PER_WS = SL_ROWS // NW

K2 = 40
NCH2 = PER_WS // K2
NBUF = 5


def _sc_gather_body(tcol_hbm, colidx_hbm, out_hbm, shared_t, idx_v, gbuf,
                    sbuf, semg, semw):
    cid = lax.axis_index("c")
    sid = lax.axis_index("s")
    wid = sid * NC + cid
    base = wid * PER_WS

    for t in range(RPT // 128):
        off = sid * RPT + t * 128
        pltpu.sync_copy(tcol_hbm.at[pl.ds(off, 128)], sbuf)
        pltpu.sync_copy(sbuf, shared_t.at[pl.ds(off, 128)])

    pltpu.sync_copy(colidx_hbm.at[wid], idx_v)
    plsc.subcore_barrier()

    def wave(w, _):
        j0 = w * NBUF
        @pl.when(w > 0)
        def _():
            for b in range(NBUF):
                pltpu.make_async_copy(
                    gbuf.at[b],
                    out_hbm.at[pl.ds(base + (j0 - NBUF + b) * K2, K2)],
                    semw.at[b]).wait()
        for b in range(NBUF):
            pltpu.async_copy(shared_t.at[idx_v.at[j0 + b]], gbuf.at[b],
                             semg.at[b])
        for b in range(NBUF):
            pltpu.make_async_copy(shared_t.at[idx_v.at[j0 + b]], gbuf.at[b],
                                  semg.at[b]).wait()
            pltpu.async_copy(gbuf.at[b],
                             out_hbm.at[pl.ds(base + (j0 + b) * K2, K2)],
                             semw.at[b])
        return 0

    lax.fori_loop(0, NCH2 // NBUF, wave, 0)

    for b in range(NBUF):
        pltpu.make_async_copy(
            gbuf.at[b],
            out_hbm.at[pl.ds(base + (NCH2 - NBUF + b) * K2, K2)],
            semw.at[b]).wait()


_sc_gather = pl.kernel(
    _sc_gather_body,
    out_type=jax.ShapeDtypeStruct((SL_ROWS, D), jnp.float32),
    mesh=_mesh,
    scratch_types=[
        pltpu.VMEM_SHARED((NPPAD, D), jnp.float32),
        pltpu.VMEM((NCH2, K2), jnp.int32),
        pltpu.VMEM((NBUF, K2, D), jnp.float32),
        pltpu.VMEM((128, D), jnp.float32),
        pltpu.SemaphoreType.DMA((NBUF,)),
        pltpu.SemaphoreType.DMA((NBUF,)),
    ],
)


BLK_P = 512
BLK_O = 6400


def _tc_tcol_body(sums_ref, cnts_ref, w_ref, b_ref, out_ref):
    s = sums_ref[0] + sums_ref[1]
    c = cnts_ref[0, :, 0:1] + cnts_ref[1, :, 0:1]
    mean = s / jnp.clip(c, 1.0)
    out_ref[...] = (jnp.dot(mean, w_ref[...] * 0.25,
                            preferred_element_type=jnp.float32)
                    + b_ref[...] * 0.25)


def _tc_tcol(sum_col, cnt_col, w_sp, b_sp):
    return pl.pallas_call(
        _tc_tcol_body,
        grid=(NPPAD // BLK_P,),
        in_specs=[
            pl.BlockSpec((NC, BLK_P, D), lambda i: (0, i, 0)),
            pl.BlockSpec((NC, BLK_P, CW), lambda i: (0, i, 0)),
            pl.BlockSpec((D, D), lambda i: (0, 0)),
            pl.BlockSpec((1, D), lambda i: (0, 0)),
        ],
        out_specs=pl.BlockSpec((BLK_P, D), lambda i: (i, 0)),
        out_shape=jax.ShapeDtypeStruct((NPPAD, D), jnp.float32),
    )(sum_col, cnt_col, w_sp, b_sp)


def _tc_trow_body(sums_ref, cnts_ref, wv_ref, bv_ref, wg_ref, bg_ref,
                  bp_ref, out_ref):
    rs = sums_ref[0] + sums_ref[1]
    rc = cnts_ref[0, :, 0:1] + cnts_ref[1, :, 0:1]
    view = (jnp.dot(rs / jnp.clip(rc, 1.0), wv_ref[...],
                    preferred_element_type=jnp.float32) + bv_ref[...])
    total = jnp.sum(rs, axis=0, keepdims=True)
    g = (jnp.dot(total * (1.0 / NNZ), wg_ref[...],
                 preferred_element_type=jnp.float32) + bg_ref[...])
    out_ref[...] = (view + g + bp_ref[...]) * 0.25


def _tc_trow(sum_row, cnt_row, w_view, b_view, w_glob, b_glob, b_proj):
    return pl.pallas_call(
        _tc_trow_body,
        out_shape=jax.ShapeDtypeStruct((NVPAD, D), jnp.float32),
    )(sum_row, cnt_row, w_view, b_view, w_glob, b_glob, b_proj)


def _tc_combine_body(vals_ref, g_ref, row_ref, trow_ref, w_ref, out_ref):
    r = row_ref[0, 0, :]
    iota = lax.broadcasted_iota(jnp.int32, (BLK_O, NVPAD), 1)
    oh = (iota == r[:, None]).astype(jnp.bfloat16)
    acc = jnp.dot(vals_ref[...].astype(jnp.bfloat16),
                  (w_ref[...] * 0.25).astype(jnp.bfloat16),
                  preferred_element_type=jnp.float32)
    acc += jnp.dot(oh, trow_ref[...].astype(jnp.bfloat16),
                   preferred_element_type=jnp.float32)
    out_ref[...] = acc + g_ref[...]


def _tc_combine_slice(s, values, g, row3, t_row, w_proj, prev_out):
    nblk = SL_ROWS // BLK_O
    off = s * nblk
    in_specs = [
        pl.BlockSpec((BLK_O, D), lambda i: (off + i, 0)),
        pl.BlockSpec((BLK_O, D), lambda i: (i, 0)),
        pl.BlockSpec((1, 1, BLK_O), lambda i: (off + i, 0, 0)),
        pl.BlockSpec((NVPAD, D), lambda i: (0, 0)),
        pl.BlockSpec((D, D), lambda i: (0, 0)),
    ]
    args = [values, g, row3, t_row, w_proj]
    kwargs = {}
    body = _tc_combine_body
    if prev_out is not None:
        in_specs.append(pl.BlockSpec(memory_space=pl.ANY))
        args.append(prev_out)
        kwargs["input_output_aliases"] = {5: 0}

        def body(v, gr, rr, tr, wr, _prev, out_ref):
            _tc_combine_body(v, gr, rr, tr, wr, out_ref)

    return pl.pallas_call(
        body,
        grid=(nblk,),
        in_specs=in_specs,
        out_specs=pl.BlockSpec((BLK_O, D), lambda i: (off + i, 0)),
        out_shape=jax.ShapeDtypeStruct((NNZ, D), jnp.float32),
        **kwargs,
    )(*args)


def kernel(values, row_idx, col_idx, W_sp, b_sp, W_view, b_view,
           W_glob, b_glob, W_proj, b_proj):
    col3 = col_idx.reshape(NW, NCH, K)
    row3 = row_idx.reshape(NW, NCH, K)

    sum_col, cnt_col, sum_row, cnt_row = _sc_segsum(values, col3, row3)
    t_col = _tc_tcol(sum_col, cnt_col, W_sp, b_sp.reshape(1, D))
    t_row = _tc_trow(sum_row, cnt_row, W_view, b_view.reshape(1, D),
                     W_glob, b_glob.reshape(1, D), b_proj.reshape(1, D))
    row3b = row_idx.reshape(NNZ // BLK_O, 1, BLK_O)
    cols = col_idx.reshape(S, NW, NCH2, K2)
    gs = [_sc_gather(t_col, cols[s]) for s in range(S)]
    out = None
    for s in range(S):
        out = _tc_combine_slice(s, values, gs[s], row3b, t_row, W_proj, out)
    return out


# speedup vs baseline: 1.0926x; 1.0159x over previous
import functools

import jax
import jax.numpy as jnp
from jax import lax
from jax.experimental import pallas as pl
from jax.experimental.pallas import tpu as pltpu
from jax.experimental.pallas import tpu_sc as plsc

N_VIEWS = 200
N_POINTS = 10000
NNZ = 320000
D = 128

NC = 2
NS = 16
NW = NC * NS
PER_W = NNZ // NW
K = 80
NCH = PER_W // K

CW = 16

NPPAD = 10240
RPT = NPPAD // NS
NVPAD = 256
RVT = 32

_mesh = plsc.VectorSubcoreMesh(
    core_axis_name="c", subcore_axis_name="s", num_cores=NC, num_subcores=NS)


def _zero_rows(zbuf, nrows, ncols16):
    z = jnp.zeros((16,), jnp.float32)

    def body(i, _):
        for c in range(ncols16):
            zbuf[i, pl.ds(16 * c, 16)] = z
        return 0

    lax.fori_loop(0, nrows, body, 0)


VR = 3
IDXR = 6


def _sc_segsum_body(values_hbm, colidx_hbm, rowidx_hbm,
                    sumcol_hbm, cntcol_hbm, sumrow_hbm, cntrow_hbm,
                    shared_col, shared_cntc, shared_row, shared_cntr,
                    idxc_v, idxr_v, vals_v, ones_v,
                    semic, semir, seml, sems):
    cid = lax.axis_index("c")
    sid = lax.axis_index("s")
    wid = sid * NC + cid
    base = wid * PER_W

    zsrc = vals_v.at[0]
    _zero_rows(zsrc, K, D // 16)
    _zero_rows(ones_v, K, 1)
    for t in range(RPT // K):
        off = sid * RPT + t * K
        pltpu.sync_copy(zsrc, shared_col.at[pl.ds(off, K)])
        pltpu.sync_copy(ones_v, shared_cntc.at[pl.ds(off, K)])

    @pl.when(sid < 8)
    def _():
        off = sid * RVT
        pltpu.sync_copy(zsrc.at[pl.ds(0, RVT)], shared_row.at[pl.ds(off, RVT)])
        pltpu.sync_copy(ones_v.at[pl.ds(0, RVT)], shared_cntr.at[pl.ds(off, RVT)])

    plsc.subcore_barrier()

    one = jnp.full((16,), 1.0, jnp.float32)

    def fill_ones(i, _):
        ones_v[i, :] = one
        return 0

    lax.fori_loop(0, K, fill_ones, 0)

    for p in range(2):
        pltpu.async_copy(colidx_hbm.at[wid, p], idxc_v.at[p], semic.at[p])
        pltpu.async_copy(rowidx_hbm.at[wid, p], idxr_v.at[p], semir.at[p])

    def step(j, _):
        bl = j % VR
        bi = j % IDXR

        @pl.when(j >= VR)
        def _():
            bio = (j - VR) % IDXR
            pltpu.make_async_copy(vals_v.at[bl],
                                  shared_col.at[idxc_v.at[bio]],
                                  sems.at[bl]).wait()
            pltpu.make_async_copy(vals_v.at[bl],
                                  shared_row.at[idxr_v.at[bio]],
                                  sems.at[bl]).wait()
            pltpu.make_async_copy(ones_v, shared_cntc.at[idxc_v.at[bio]],
                                  sems.at[bl]).wait()
            pltpu.make_async_copy(ones_v, shared_cntr.at[idxr_v.at[bio]],
                                  sems.at[bl]).wait()

        pltpu.async_copy(values_hbm.at[pl.ds(base + j * K, K)],
                         vals_v.at[bl], seml.at[bl])

        @pl.when(j + 2 < NCH)
        def _():
            b2 = (j + 2) % IDXR
            pltpu.async_copy(colidx_hbm.at[wid, j + 2], idxc_v.at[b2],
                             semic.at[b2])
            pltpu.async_copy(rowidx_hbm.at[wid, j + 2], idxr_v.at[b2],
                             semir.at[b2])

        pltpu.make_async_copy(colidx_hbm.at[wid, j], idxc_v.at[bi],
                              semic.at[bi]).wait()
        pltpu.make_async_copy(rowidx_hbm.at[wid, j], idxr_v.at[bi],
                              semir.at[bi]).wait()
        pltpu.make_async_copy(values_hbm.at[pl.ds(base + j * K, K)],
                              vals_v.at[bl], seml.at[bl]).wait()

        pltpu.async_copy(vals_v.at[bl], shared_col.at[idxc_v.at[bi]],
                         sems.at[bl], add=True)
        pltpu.async_copy(vals_v.at[bl], shared_row.at[idxr_v.at[bi]],
                         sems.at[bl], add=True)
        pltpu.async_copy(ones_v, shared_cntc.at[idxc_v.at[bi]],
                         sems.at[bl], add=True)
        pltpu.async_copy(ones_v, shared_cntr.at[idxr_v.at[bi]],
                         sems.at[bl], add=True)
        return 0

    lax.fori_loop(0, NCH, step, 0)

    for d in range(VR):
        j = NCH - VR + d
        bl = j % VR
        pltpu.make_async_copy(vals_v.at[bl], shared_col.at[idxc_v.at[0]],
                              sems.at[bl]).wait()
        pltpu.make_async_copy(vals_v.at[bl], shared_row.at[idxr_v.at[0]],
                              sems.at[bl]).wait()
        pltpu.make_async_copy(ones_v, shared_cntc.at[idxc_v.at[0]],
                              sems.at[bl]).wait()
        pltpu.make_async_copy(ones_v, shared_cntr.at[idxr_v.at[0]],
                              sems.at[bl]).wait()

    plsc.subcore_barrier()

    for t in range(RPT // 128):
        off = sid * RPT + t * 128
        pltpu.sync_copy(shared_col.at[pl.ds(off, 128)],
                        sumcol_hbm.at[cid, pl.ds(off, 128)])
    for t in range(RPT // 128):
        off = sid * RPT + t * 128
        pltpu.sync_copy(shared_cntc.at[pl.ds(off, 128)],
                        cntcol_hbm.at[cid, pl.ds(off, 128)])

    @pl.when(sid < 8)
    def _():
        off = sid * RVT
        pltpu.sync_copy(shared_row.at[pl.ds(off, RVT)],
                        sumrow_hbm.at[cid, pl.ds(off, RVT)])
        pltpu.sync_copy(shared_cntr.at[pl.ds(off, RVT)],
                        cntrow_hbm.at[cid, pl.ds(off, RVT)])


_sc_segsum = pl.kernel(
    _sc_segsum_body,
    out_type=[
        jax.ShapeDtypeStruct((NC, NPPAD, D), jnp.float32),
        jax.ShapeDtypeStruct((NC, NPPAD, CW), jnp.float32),
        jax.ShapeDtypeStruct((NC, NVPAD, D), jnp.float32),
        jax.ShapeDtypeStruct((NC, NVPAD, CW), jnp.float32),
    ],
    mesh=_mesh,
    scratch_types=[
        pltpu.VMEM_SHARED((NPPAD, D), jnp.float32),
        pltpu.VMEM_SHARED((NPPAD, CW), jnp.float32),
        pltpu.VMEM_SHARED((NVPAD, D), jnp.float32),
        pltpu.VMEM_SHARED((NVPAD, CW), jnp.float32),
        pltpu.VMEM((IDXR, K), jnp.int32),
        pltpu.VMEM((IDXR, K), jnp.int32),
        pltpu.VMEM((VR, K, D), jnp.float32),
        pltpu.VMEM((K, CW), jnp.float32),
        pltpu.SemaphoreType.DMA((IDXR,)),
        pltpu.SemaphoreType.DMA((IDXR,)),
        pltpu.SemaphoreType.DMA((VR,)),
        pltpu.SemaphoreType.DMA((VR,)),
    ],
    compiler_params=pltpu.CompilerParams(use_tc_tiling_on_sc=False),
)


S = 5
SL_ROWS = NNZ // S
PER_WS = SL_ROWS // NW

K2 = 40
NCH2 = PER_WS // K2
NBUF = 5


def _sc_gather_body(tcol_hbm, colidx_hbm, out_hbm, shared_t, idx_v, gbuf,
                    sbuf, semg, semw):
    cid = lax.axis_index("c")
    sid = lax.axis_index("s")
    wid = sid * NC + cid
    base = wid * PER_WS

    for t in range(RPT // 128):
        off = sid * RPT + t * 128
        pltpu.sync_copy(tcol_hbm.at[pl.ds(off, 128)], sbuf)
        pltpu.sync_copy(sbuf, shared_t.at[pl.ds(off, 128)])

    pltpu.sync_copy(colidx_hbm.at[wid], idx_v)
    plsc.subcore_barrier()

    def wave(w, _):
        j0 = w * NBUF
        @pl.when(w > 0)
        def _():
            for b in range(NBUF):
                pltpu.make_async_copy(
                    gbuf.at[b],
                    out_hbm.at[pl.ds(base + (j0 - NBUF + b) * K2, K2)],
                    semw.at[b]).wait()
        for b in range(NBUF):
            pltpu.async_copy(shared_t.at[idx_v.at[j0 + b]], gbuf.at[b],
                             semg.at[b])
        for b in range(NBUF):
            pltpu.make_async_copy(shared_t.at[idx_v.at[j0 + b]], gbuf.at[b],
                                  semg.at[b]).wait()
            pltpu.async_copy(gbuf.at[b],
                             out_hbm.at[pl.ds(base + (j0 + b) * K2, K2)],
                             semw.at[b])
        return 0

    lax.fori_loop(0, NCH2 // NBUF, wave, 0)

    for b in range(NBUF):
        pltpu.make_async_copy(
            gbuf.at[b],
            out_hbm.at[pl.ds(base + (NCH2 - NBUF + b) * K2, K2)],
            semw.at[b]).wait()


_sc_gather = pl.kernel(
    _sc_gather_body,
    out_type=jax.ShapeDtypeStruct((SL_ROWS, D), jnp.float32),
    mesh=_mesh,
    scratch_types=[
        pltpu.VMEM_SHARED((NPPAD, D), jnp.float32),
        pltpu.VMEM((NCH2, K2), jnp.int32),
        pltpu.VMEM((NBUF, K2, D), jnp.float32),
        pltpu.VMEM((128, D), jnp.float32),
        pltpu.SemaphoreType.DMA((NBUF,)),
        pltpu.SemaphoreType.DMA((NBUF,)),
    ],
)


BLK_P = 2048
BLK_O = 6400


def _tc_tcol_body(sums_ref, cnts_ref, w_ref, b_ref, out_ref):
    s = sums_ref[0] + sums_ref[1]
    c = cnts_ref[0, :, 0:1] + cnts_ref[1, :, 0:1]
    mean = s / jnp.clip(c, 1.0)
    out_ref[...] = (jnp.dot(mean, w_ref[...] * 0.25,
                            preferred_element_type=jnp.float32)
                    + b_ref[...] * 0.25)


def _tc_tcol(sum_col, cnt_col, w_sp, b_sp):
    return pl.pallas_call(
        _tc_tcol_body,
        grid=(NPPAD // BLK_P,),
        in_specs=[
            pl.BlockSpec((NC, BLK_P, D), lambda i: (0, i, 0)),
            pl.BlockSpec((NC, BLK_P, CW), lambda i: (0, i, 0)),
            pl.BlockSpec((D, D), lambda i: (0, 0)),
            pl.BlockSpec((1, D), lambda i: (0, 0)),
        ],
        out_specs=pl.BlockSpec((BLK_P, D), lambda i: (i, 0)),
        out_shape=jax.ShapeDtypeStruct((NPPAD, D), jnp.float32),
    )(sum_col, cnt_col, w_sp, b_sp)


def _tc_trow_body(sums_ref, cnts_ref, wv_ref, bv_ref, wg_ref, bg_ref,
                  bp_ref, out_ref):
    rs = sums_ref[0] + sums_ref[1]
    rc = cnts_ref[0, :, 0:1] + cnts_ref[1, :, 0:1]
    view = (jnp.dot(rs / jnp.clip(rc, 1.0), wv_ref[...],
                    preferred_element_type=jnp.float32) + bv_ref[...])
    total = jnp.sum(rs, axis=0, keepdims=True)
    g = (jnp.dot(total * (1.0 / NNZ), wg_ref[...],
                 preferred_element_type=jnp.float32) + bg_ref[...])
    out_ref[...] = (view + g + bp_ref[...]) * 0.25


def _tc_trow(sum_row, cnt_row, w_view, b_view, w_glob, b_glob, b_proj):
    return pl.pallas_call(
        _tc_trow_body,
        out_shape=jax.ShapeDtypeStruct((NVPAD, D), jnp.float32),
    )(sum_row, cnt_row, w_view, b_view, w_glob, b_glob, b_proj)


def _tc_combine_body(vals_ref, g_ref, row_ref, trow_ref, w_ref, out_ref):
    r = row_ref[0, 0, :]
    iota = lax.broadcasted_iota(jnp.int32, (BLK_O, NVPAD), 1)
    oh = (iota == r[:, None]).astype(jnp.bfloat16)
    acc = jnp.dot(vals_ref[...].astype(jnp.bfloat16),
                  (w_ref[...] * 0.25).astype(jnp.bfloat16),
                  preferred_element_type=jnp.float32)
    acc += jnp.dot(oh, trow_ref[...].astype(jnp.bfloat16),
                   preferred_element_type=jnp.float32)
    out_ref[...] = acc + g_ref[...]


def _tc_combine_slice(s, values, g, row3, t_row, w_proj, prev_out):
    nblk = SL_ROWS // BLK_O
    off = s * nblk
    in_specs = [
        pl.BlockSpec((BLK_O, D), lambda i: (off + i, 0)),
        pl.BlockSpec((BLK_O, D), lambda i: (i, 0)),
        pl.BlockSpec((1, 1, BLK_O), lambda i: (off + i, 0, 0)),
        pl.BlockSpec((NVPAD, D), lambda i: (0, 0)),
        pl.BlockSpec((D, D), lambda i: (0, 0)),
    ]
    args = [values, g, row3, t_row, w_proj]
    kwargs = {}
    body = _tc_combine_body
    if prev_out is not None:
        in_specs.append(pl.BlockSpec(memory_space=pl.ANY))
        args.append(prev_out)
        kwargs["input_output_aliases"] = {5: 0}

        def body(v, gr, rr, tr, wr, _prev, out_ref):
            _tc_combine_body(v, gr, rr, tr, wr, out_ref)

    return pl.pallas_call(
        body,
        grid=(nblk,),
        in_specs=in_specs,
        out_specs=pl.BlockSpec((BLK_O, D), lambda i: (off + i, 0)),
        out_shape=jax.ShapeDtypeStruct((NNZ, D), jnp.float32),
        **kwargs,
    )(*args)


def kernel(values, row_idx, col_idx, W_sp, b_sp, W_view, b_view,
           W_glob, b_glob, W_proj, b_proj):
    col3 = col_idx.reshape(NW, NCH, K)
    row3 = row_idx.reshape(NW, NCH, K)

    sum_col, cnt_col, sum_row, cnt_row = _sc_segsum(values, col3, row3)
    t_col = _tc_tcol(sum_col, cnt_col, W_sp, b_sp.reshape(1, D))
    t_row = _tc_trow(sum_row, cnt_row, W_view, b_view.reshape(1, D),
                     W_glob, b_glob.reshape(1, D), b_proj.reshape(1, D))
    row3b = row_idx.reshape(NNZ // BLK_O, 1, BLK_O)
    cols = col_idx.reshape(S, NW, NCH2, K2)
    gs = [_sc_gather(t_col, cols[s]) for s in range(S)]
    out = None
    for s in range(S):
        out = _tc_combine_slice(s, values, gs[s], row3b, t_row, W_proj, out)
    return out


# speedup vs baseline: 1.0937x; 1.0010x over previous
import jax
import jax.numpy as jnp
from jax import lax
from jax.experimental import pallas as pl
from jax.experimental.pallas import tpu as pltpu
from jax.experimental.pallas import tpu_sc as plsc

N_VIEWS = 200
N_POINTS = 10000
NNZ = 320000
D = 128

NC = 2
NS = 16
NW = NC * NS
PER_W = NNZ // NW
K = 80
NCH = PER_W // K

CW = 16

NPPAD = 10240
RPT = NPPAD // NS
NVPAD = 256
RVT = 32

_mesh = plsc.VectorSubcoreMesh(
    core_axis_name="c", subcore_axis_name="s", num_cores=NC, num_subcores=NS)


def _zero_rows(zbuf, nrows, ncols16):
    z = jnp.zeros((16,), jnp.float32)

    def body(i, _):
        for c in range(ncols16):
            zbuf[i, pl.ds(16 * c, 16)] = z
        return 0

    lax.fori_loop(0, nrows, body, 0)


VR = 3
IDXR = 6


def _sc_segsum_body(values_hbm, colidx_hbm, rowidx_hbm,
                    sumcol_hbm, cntcol_hbm, sumrow_hbm, cntrow_hbm,
                    shared_col, shared_cntc, shared_row, shared_cntr,
                    idxc_v, idxr_v, vals_v, ones_v,
                    semic, semir, seml, sems):
    cid = lax.axis_index("c")
    sid = lax.axis_index("s")
    wid = sid * NC + cid
    base = wid * PER_W

    zsrc = vals_v.at[0]
    _zero_rows(zsrc, K, D // 16)
    _zero_rows(ones_v, K, 1)
    for t in range(RPT // K):
        off = sid * RPT + t * K
        pltpu.sync_copy(zsrc, shared_col.at[pl.ds(off, K)])
        pltpu.sync_copy(ones_v, shared_cntc.at[pl.ds(off, K)])

    @pl.when(sid < 8)
    def _():
        off = sid * RVT
        pltpu.sync_copy(zsrc.at[pl.ds(0, RVT)], shared_row.at[pl.ds(off, RVT)])
        pltpu.sync_copy(ones_v.at[pl.ds(0, RVT)], shared_cntr.at[pl.ds(off, RVT)])

    plsc.subcore_barrier()

    one = jnp.full((16,), 1.0, jnp.float32)

    def fill_ones(i, _):
        ones_v[i, :] = one
        return 0

    lax.fori_loop(0, K, fill_ones, 0)

    for p in range(2):
        pltpu.async_copy(colidx_hbm.at[wid, p], idxc_v.at[p], semic.at[p])
        pltpu.async_copy(rowidx_hbm.at[wid, p], idxr_v.at[p], semir.at[p])

    def step(j, _):
        bl = j % VR
        bi = j % IDXR

        @pl.when(j >= VR)
        def _():
            bio = (j - VR) % IDXR
            pltpu.make_async_copy(vals_v.at[bl],
                                  shared_col.at[idxc_v.at[bio]],
                                  sems.at[bl]).wait()
            pltpu.make_async_copy(vals_v.at[bl],
                                  shared_row.at[idxr_v.at[bio]],
                                  sems.at[bl]).wait()
            pltpu.make_async_copy(ones_v, shared_cntc.at[idxc_v.at[bio]],
                                  sems.at[bl]).wait()
            pltpu.make_async_copy(ones_v, shared_cntr.at[idxr_v.at[bio]],
                                  sems.at[bl]).wait()

        pltpu.async_copy(values_hbm.at[pl.ds(base + j * K, K)],
                         vals_v.at[bl], seml.at[bl])

        @pl.when(j + 2 < NCH)
        def _():
            b2 = (j + 2) % IDXR
            pltpu.async_copy(colidx_hbm.at[wid, j + 2], idxc_v.at[b2],
                             semic.at[b2])
            pltpu.async_copy(rowidx_hbm.at[wid, j + 2], idxr_v.at[b2],
                             semir.at[b2])

        pltpu.make_async_copy(colidx_hbm.at[wid, j], idxc_v.at[bi],
                              semic.at[bi]).wait()
        pltpu.make_async_copy(rowidx_hbm.at[wid, j], idxr_v.at[bi],
                              semir.at[bi]).wait()
        pltpu.make_async_copy(values_hbm.at[pl.ds(base + j * K, K)],
                              vals_v.at[bl], seml.at[bl]).wait()

        pltpu.async_copy(vals_v.at[bl], shared_col.at[idxc_v.at[bi]],
                         sems.at[bl], add=True)
        pltpu.async_copy(vals_v.at[bl], shared_row.at[idxr_v.at[bi]],
                         sems.at[bl], add=True)
        pltpu.async_copy(ones_v, shared_cntc.at[idxc_v.at[bi]],
                         sems.at[bl], add=True)
        pltpu.async_copy(ones_v, shared_cntr.at[idxr_v.at[bi]],
                         sems.at[bl], add=True)
        return 0

    lax.fori_loop(0, NCH, step, 0)

    for d in range(VR):
        j = NCH - VR + d
        bl = j % VR
        pltpu.make_async_copy(vals_v.at[bl], shared_col.at[idxc_v.at[0]],
                              sems.at[bl]).wait()
        pltpu.make_async_copy(vals_v.at[bl], shared_row.at[idxr_v.at[0]],
                              sems.at[bl]).wait()
        pltpu.make_async_copy(ones_v, shared_cntc.at[idxc_v.at[0]],
                              sems.at[bl]).wait()
        pltpu.make_async_copy(ones_v, shared_cntr.at[idxr_v.at[0]],
                              sems.at[bl]).wait()

    plsc.subcore_barrier()

    for t in range(RPT // 128):
        off = sid * RPT + t * 128
        pltpu.sync_copy(shared_col.at[pl.ds(off, 128)],
                        sumcol_hbm.at[cid, pl.ds(off, 128)])
    for t in range(RPT // 128):
        off = sid * RPT + t * 128
        pltpu.sync_copy(shared_cntc.at[pl.ds(off, 128)],
                        cntcol_hbm.at[cid, pl.ds(off, 128)])

    @pl.when(sid < 8)
    def _():
        off = sid * RVT
        pltpu.sync_copy(shared_row.at[pl.ds(off, RVT)],
                        sumrow_hbm.at[cid, pl.ds(off, RVT)])
        pltpu.sync_copy(shared_cntr.at[pl.ds(off, RVT)],
                        cntrow_hbm.at[cid, pl.ds(off, RVT)])


_sc_segsum = pl.kernel(
    _sc_segsum_body,
    out_type=[
        jax.ShapeDtypeStruct((NC, NPPAD, D), jnp.float32),
        jax.ShapeDtypeStruct((NC, NPPAD, CW), jnp.float32),
        jax.ShapeDtypeStruct((NC, NVPAD, D), jnp.float32),
        jax.ShapeDtypeStruct((NC, NVPAD, CW), jnp.float32),
    ],
    mesh=_mesh,
    scratch_types=[
        pltpu.VMEM_SHARED((NPPAD, D), jnp.float32),
        pltpu.VMEM_SHARED((NPPAD, CW), jnp.float32),
        pltpu.VMEM_SHARED((NVPAD, D), jnp.float32),
        pltpu.VMEM_SHARED((NVPAD, CW), jnp.float32),
        pltpu.VMEM((IDXR, K), jnp.int32),
        pltpu.VMEM((IDXR, K), jnp.int32),
        pltpu.VMEM((VR, K, D), jnp.float32),
        pltpu.VMEM((K, CW), jnp.float32),
        pltpu.SemaphoreType.DMA((IDXR,)),
        pltpu.SemaphoreType.DMA((IDXR,)),
        pltpu.SemaphoreType.DMA((VR,)),
        pltpu.SemaphoreType.DMA((VR,)),
    ],
    compiler_params=pltpu.CompilerParams(use_tc_tiling_on_sc=False),
)


S = 5
SL_ROWS = NNZ // S
PER_WS = SL_ROWS // NW

K2 = 40
NCH2 = PER_WS // K2
NBUF = 5


def _sc_gather_body(tcol_hbm, colidx_hbm, out_hbm, shared_t, idx_v, gbuf,
                    sbuf, semg, semw):
    cid = lax.axis_index("c")
    sid = lax.axis_index("s")
    wid = sid * NC + cid
    base = wid * PER_WS

    for t in range(RPT // 128):
        off = sid * RPT + t * 128
        pltpu.sync_copy(tcol_hbm.at[pl.ds(off, 128)], sbuf)
        pltpu.sync_copy(sbuf, shared_t.at[pl.ds(off, 128)])

    pltpu.sync_copy(colidx_hbm.at[wid], idx_v)
    plsc.subcore_barrier()

    def wave(w, _):
        j0 = w * NBUF
        @pl.when(w > 0)
        def _():
            for b in range(NBUF):
                pltpu.make_async_copy(
                    gbuf.at[b],
                    out_hbm.at[pl.ds(base + (j0 - NBUF + b) * K2, K2)],
                    semw.at[b]).wait()
        for b in range(NBUF):
            pltpu.async_copy(shared_t.at[idx_v.at[j0 + b]], gbuf.at[b],
                             semg.at[b])
        for b in range(NBUF):
            pltpu.make_async_copy(shared_t.at[idx_v.at[j0 + b]], gbuf.at[b],
                                  semg.at[b]).wait()
            pltpu.async_copy(gbuf.at[b],
                             out_hbm.at[pl.ds(base + (j0 + b) * K2, K2)],
                             semw.at[b])
        return 0

    lax.fori_loop(0, NCH2 // NBUF, wave, 0)

    for b in range(NBUF):
        pltpu.make_async_copy(
            gbuf.at[b],
            out_hbm.at[pl.ds(base + (NCH2 - NBUF + b) * K2, K2)],
            semw.at[b]).wait()


_sc_gather = pl.kernel(
    _sc_gather_body,
    out_type=jax.ShapeDtypeStruct((SL_ROWS, D), jnp.float32),
    mesh=_mesh,
    scratch_types=[
        pltpu.VMEM_SHARED((NPPAD, D), jnp.float32),
        pltpu.VMEM((NCH2, K2), jnp.int32),
        pltpu.VMEM((NBUF, K2, D), jnp.float32),
        pltpu.VMEM((128, D), jnp.float32),
        pltpu.SemaphoreType.DMA((NBUF,)),
        pltpu.SemaphoreType.DMA((NBUF,)),
    ],
)


BLK_P = 2048
BLK_O = 6400


def _tc_tcol_body(sums_ref, cnts_ref, w_ref, b_ref, out_ref):
    s = sums_ref[0] + sums_ref[1]
    c = cnts_ref[0, :, 0:1] + cnts_ref[1, :, 0:1]
    mean = s / jnp.clip(c, 1.0)
    out_ref[...] = (jnp.dot(mean, w_ref[...] * 0.25,
                            preferred_element_type=jnp.float32)
                    + b_ref[...] * 0.25)


def _tc_tcol(sum_col, cnt_col, w_sp, b_sp):
    return pl.pallas_call(
        _tc_tcol_body,
        grid=(NPPAD // BLK_P,),
        in_specs=[
            pl.BlockSpec((NC, BLK_P, D), lambda i: (0, i, 0)),
            pl.BlockSpec((NC, BLK_P, CW), lambda i: (0, i, 0)),
            pl.BlockSpec((D, D), lambda i: (0, 0)),
            pl.BlockSpec((1, D), lambda i: (0, 0)),
        ],
        out_specs=pl.BlockSpec((BLK_P, D), lambda i: (i, 0)),
        out_shape=jax.ShapeDtypeStruct((NPPAD, D), jnp.float32),
    )(sum_col, cnt_col, w_sp, b_sp)


def _tc_trow_body(sums_ref, cnts_ref, wv_ref, bv_ref, wg_ref, bg_ref,
                  bp_ref, out_ref):
    rs = sums_ref[0] + sums_ref[1]
    rc = cnts_ref[0, :, 0:1] + cnts_ref[1, :, 0:1]
    view = (jnp.dot(rs / jnp.clip(rc, 1.0), wv_ref[...],
                    preferred_element_type=jnp.float32) + bv_ref[...])
    total = jnp.sum(rs, axis=0, keepdims=True)
    g = (jnp.dot(total * (1.0 / NNZ), wg_ref[...],
                 preferred_element_type=jnp.float32) + bg_ref[...])
    out_ref[...] = (view + g + bp_ref[...]) * 0.25


def _tc_trow(sum_row, cnt_row, w_view, b_view, w_glob, b_glob, b_proj):
    return pl.pallas_call(
        _tc_trow_body,
        out_shape=jax.ShapeDtypeStruct((NVPAD, D), jnp.float32),
    )(sum_row, cnt_row, w_view, b_view, w_glob, b_glob, b_proj)


def _tc_combine_body(vals_ref, g_ref, row_ref, trow_ref, w_ref, out_ref):
    r = row_ref[0, 0, :]
    iota = lax.broadcasted_iota(jnp.int32, (BLK_O, NVPAD), 1)
    oh = (iota == r[:, None]).astype(jnp.bfloat16)
    acc = jnp.dot(vals_ref[...].astype(jnp.bfloat16),
                  (w_ref[...] * 0.25).astype(jnp.bfloat16),
                  preferred_element_type=jnp.float32)
    acc += jnp.dot(oh, trow_ref[...].astype(jnp.bfloat16),
                   preferred_element_type=jnp.float32)
    out_ref[...] = acc + g_ref[...]


def _tc_combine_slice(s, values, g, row3, t_row, w_proj, prev_out):
    nblk = SL_ROWS // BLK_O
    off = s * nblk
    in_specs = [
        pl.BlockSpec((BLK_O, D), lambda i: (off + i, 0)),
        pl.BlockSpec((BLK_O, D), lambda i: (i, 0)),
        pl.BlockSpec((1, 1, BLK_O), lambda i: (off + i, 0, 0)),
        pl.BlockSpec((NVPAD, D), lambda i: (0, 0)),
        pl.BlockSpec((D, D), lambda i: (0, 0)),
    ]
    args = [values, g, row3, t_row, w_proj]
    kwargs = {}
    body = _tc_combine_body
    if prev_out is not None:
        in_specs.append(pl.BlockSpec(memory_space=pl.ANY))
        args.append(prev_out)
        kwargs["input_output_aliases"] = {5: 0}

        def body(v, gr, rr, tr, wr, _prev, out_ref):
            _tc_combine_body(v, gr, rr, tr, wr, out_ref)

    return pl.pallas_call(
        body,
        grid=(nblk,),
        in_specs=in_specs,
        out_specs=pl.BlockSpec((BLK_O, D), lambda i: (off + i, 0)),
        out_shape=jax.ShapeDtypeStruct((NNZ, D), jnp.float32),
        **kwargs,
    )(*args)


def kernel(values, row_idx, col_idx, W_sp, b_sp, W_view, b_view,
           W_glob, b_glob, W_proj, b_proj):
    col3 = col_idx.reshape(NW, NCH, K)
    row3 = row_idx.reshape(NW, NCH, K)

    sum_col, cnt_col, sum_row, cnt_row = _sc_segsum(values, col3, row3)
    t_col = _tc_tcol(sum_col, cnt_col, W_sp, b_sp.reshape(1, D))
    t_row = _tc_trow(sum_row, cnt_row, W_view, b_view.reshape(1, D),
                     W_glob, b_glob.reshape(1, D), b_proj.reshape(1, D))
    row3b = row_idx.reshape(NNZ // BLK_O, 1, BLK_O)
    cols = col_idx.reshape(S, NW, NCH2, K2)
    gs = [_sc_gather(t_col, cols[s]) for s in range(S)]
    out = None
    for s in range(S):
        out = _tc_combine_slice(s, values, gs[s], row3b, t_row, W_proj, out)
    return out
